# baseline (speedup 1.0000x reference)
import functools

import jax
import jax.numpy as jnp
from jax import lax
from jax.experimental import pallas as pl
from jax.experimental.pallas import tpu as pltpu
from jax.experimental.pallas import tpu_sc as plsc

B = 4096
M = 200
R = 100000
A = 64
S = 64
H = 128
PAD = 0

L = 16
NC, NS = 2, 16
NW = NC * NS
BPW = B // NW
MP = 208
MH = MP // 2

_MESH = plsc.VectorSubcoreMesh(
    core_axis_name="c", subcore_axis_name="s", num_cores=NC, num_subcores=NS)
_SC_PARAMS = pltpu.CompilerParams(
    use_tc_tiling_on_sc=False, needs_layout_passes=False)


def _worker_id():
    return lax.axis_index("s") * NC + lax.axis_index("c")


@functools.partial(
    pl.kernel,
    out_type=(jax.ShapeDtypeStruct((B, A), jnp.bfloat16),
              jax.ShapeDtypeStruct((B, A), jnp.bfloat16)),
    mesh=_MESH,
    scratch_types=[
        pltpu.VMEM((BPW,), jnp.int32),
        pltpu.VMEM((BPW,), jnp.int32),
        pltpu.VMEM((BPW, A), jnp.bfloat16),
        pltpu.VMEM((BPW, A), jnp.bfloat16),
        pltpu.SemaphoreType.DMA,
        pltpu.SemaphoreType.DMA,
    ],
    compiler_params=_SC_PARAMS,
)
def _embed_gather(table, prev_rel, queries, pe_out, qe_out,
                  idx1_v, idx2_v, rows1_v, rows2_v, sem1, sem2):
    base = _worker_id() * BPW
    pltpu.sync_copy(prev_rel.at[pl.ds(base, BPW)], idx1_v)
    pltpu.sync_copy(queries.at[pl.ds(base, BPW)], idx2_v)
    g1 = pltpu.async_copy(table.at[idx1_v], rows1_v, sem1)
    g2 = pltpu.async_copy(table.at[idx2_v], rows2_v, sem2)
    g1.wait()
    g2.wait()
    pltpu.sync_copy(rows1_v, pe_out.at[pl.ds(base, BPW)])
    pltpu.sync_copy(rows2_v, qe_out.at[pl.ds(base, BPW)])


def _lstm_mlp_body(pe_ref, qe_ref, h_ref, c_ref, wih_t_ref, whh_t_ref, b_ref,
                   w1_t_ref, b1_ref, w2_t_ref, b2_ref,
                   h2_ref, c2_ref, out_ref):
    x = pe_ref[...].astype(jnp.float32)
    h = h_ref[...]
    gates = (jnp.dot(x, wih_t_ref[...], preferred_element_type=jnp.float32)
             + jnp.dot(h, whh_t_ref[...], preferred_element_type=jnp.float32)
             + b_ref[...])
    i = jax.nn.sigmoid(gates[:, 0:S])
    f = jax.nn.sigmoid(gates[:, S:2 * S])
    g = jnp.tanh(gates[:, 2 * S:3 * S])
    o = jax.nn.sigmoid(gates[:, 3 * S:4 * S])
    c2 = f * c_ref[...] + i * g
    h2 = o * jnp.tanh(c2)
    sq = jnp.concatenate([h2, qe_ref[...].astype(jnp.float32)], axis=-1)
    hid = jax.nn.relu(
        jnp.dot(sq, w1_t_ref[...], preferred_element_type=jnp.float32)
        + b1_ref[...])
    out = jax.nn.relu(
        jnp.dot(hid, w2_t_ref[...], preferred_element_type=jnp.float32)
        + b2_ref[...])
    h2_ref[...] = h2
    c2_ref[...] = c2
    out_ref[...] = out.astype(jnp.bfloat16)


def _lstm_mlp(pe, qe, h, c, wih_t, whh_t, b, w1_t, b1, w2_t, b2):
    return pl.pallas_call(
        _lstm_mlp_body,
        out_shape=(jax.ShapeDtypeStruct((B, S), jnp.float32),
                   jax.ShapeDtypeStruct((B, S), jnp.float32),
                   jax.ShapeDtypeStruct((B, A), jnp.bfloat16)),
    )(pe, qe, h, c, wih_t, whh_t, b, w1_t, b1, w2_t, b2)


@functools.partial(
    pl.kernel,
    out_type=jax.ShapeDtypeStruct((B, MP), jnp.float32),
    mesh=_MESH,
    scratch_types=[
        pltpu.VMEM((BPW, MP), jnp.int32),
        pltpu.VMEM((BPW, A), jnp.bfloat16),
        pltpu.VMEM((MP, A), jnp.bfloat16),
        pltpu.VMEM((MP, A), jnp.bfloat16),
        pltpu.VMEM((BPW, MP), jnp.float32),
        pltpu.SemaphoreType.DMA,
        pltpu.SemaphoreType.DMA,
    ],
    compiler_params=_SC_PARAMS,
)
def _scores_kernel(table, ids, outv, scores_out,
                   ids_v, out_v, rows0_v, rows1_v, sc_v, sem0, sem1):
    base = _worker_id() * BPW
    pltpu.sync_copy(ids.at[pl.ds(base, BPW)], ids_v)
    pltpu.sync_copy(outv.at[pl.ds(base, BPW)], out_v)

    def issue(i, buf, sem):
        pltpu.async_copy(
            table.at[ids_v.at[i, pl.ds(0, MH)]], buf.at[pl.ds(0, MH)], sem)
        pltpu.async_copy(
            table.at[ids_v.at[i, pl.ds(MH, MH)]], buf.at[pl.ds(MH, MH)], sem)

    def wait(buf, sem):
        pltpu.make_async_copy(
            table.at[ids_v.at[0, pl.ds(0, MH)]], buf.at[pl.ds(0, MH)],
            sem).wait()
        pltpu.make_async_copy(
            table.at[ids_v.at[0, pl.ds(MH, MH)]], buf.at[pl.ds(MH, MH)],
            sem).wait()

    lane = lax.iota(jnp.int32, L)

    def compute(i, buf):
        o0 = out_v[i, pl.ds(0, 2 * L)]
        o1 = out_v[i, pl.ds(2 * L, 2 * L)]

        def blk_body(bi, c2):
            sv = [jnp.zeros((L,), jnp.float32) for _ in range(4)]
            for j in range(L):
                m = bi * L + j
                p = (buf[m, pl.ds(0, 2 * L)] * o0
                     + buf[m, pl.ds(2 * L, 2 * L)] * o1)
                pa, pb = plsc.unpack(
                    p, format=plsc.PackFormat.INTERLEAVED,
                    preferred_element_type=jnp.float32)
                q = j % 4
                sv[q] = jnp.where(lane == j, jnp.sum(pa + pb), sv[q])
            idv = ids_v[i, pl.ds(bi * L, L)]
            merged = (sv[0] + sv[1]) + (sv[2] + sv[3])
            sc_v[i, pl.ds(bi * L, L)] = jnp.where(
                idv == PAD, -99999.0, merged)
            return c2

        lax.fori_loop(0, MP // L, blk_body, 0)

    issue(0, rows0_v, sem0)

    def pair_body(k, carry):
        i0 = 2 * k
        issue(i0 + 1, rows1_v, sem1)
        wait(rows0_v, sem0)
        compute(i0, rows0_v)
        issue(jnp.minimum(i0 + 2, BPW - 1), rows0_v, sem0)
        wait(rows1_v, sem1)
        compute(i0 + 1, rows1_v)
        return carry

    lax.fori_loop(0, BPW // 2, pair_body, 0)
    wait(rows0_v, sem0)
    pltpu.sync_copy(sc_v, scores_out.at[pl.ds(base, BPW)])


def _logits_body(sc_ref, logits_ref):
    s = sc_ref[...][:, :M]
    mx = jnp.max(s, axis=-1, keepdims=True)
    lse = jnp.log(jnp.sum(jnp.exp(s - mx), axis=-1, keepdims=True)) + mx
    logits_ref[...] = s - lse


def _logits(scores):
    return pl.pallas_call(
        _logits_body,
        out_shape=jax.ShapeDtypeStruct((B, M), jnp.float32),
    )(scores)


def kernel(prev_state_h, prev_state_c, prev_relation, actions_id, queries,
           rel_emb, W_ih, W_hh, b_ih, b_hh, mlp1_W, mlp1_b, mlp2_W, mlp2_b):
    rel_ids = actions_id[:, :, 0]
    ent_ids = actions_id[:, :, 1]
    ids_pad = jnp.pad(rel_ids, ((0, 0), (0, MP - M)))
    t16 = rel_emb.astype(jnp.bfloat16)

    pe, qe = _embed_gather(t16, prev_relation, queries)

    b_all = (b_ih + b_hh).reshape(1, 4 * S)
    h2, c2, outv = _lstm_mlp(
        pe, qe, prev_state_h, prev_state_c,
        W_ih.T, W_hh.T, b_all,
        mlp1_W.T, mlp1_b.reshape(1, H), mlp2_W.T, mlp2_b.reshape(1, A))

    scores = _scores_kernel(t16, ids_pad, outv)
    logits = _logits(scores)
    return (logits, rel_ids, ent_ids, h2, c2)


# speedup vs baseline: 2.2074x; 2.2074x over previous
import functools

import jax
import jax.numpy as jnp
from jax import lax
from jax.experimental import pallas as pl
from jax.experimental.pallas import tpu as pltpu
from jax.experimental.pallas import tpu_sc as plsc

B = 4096
M = 200
R = 100000
A = 64
S = 64
H = 128
PAD = 0

L = 16
NC, NS = 2, 16
NW = NC * NS
BPW = B // NW
MP = 208
MH = MP // 2
MH2 = M - MH
NBUF = 4

_MESH = plsc.VectorSubcoreMesh(
    core_axis_name="c", subcore_axis_name="s", num_cores=NC, num_subcores=NS)
_SC_PARAMS = pltpu.CompilerParams(
    use_tc_tiling_on_sc=False, needs_layout_passes=False)


def _worker_id():
    return lax.axis_index("s") * NC + lax.axis_index("c")


@functools.partial(
    pl.kernel,
    out_type=(jax.ShapeDtypeStruct((B, A), jnp.float32),
              jax.ShapeDtypeStruct((B, A), jnp.float32)),
    mesh=_MESH,
    scratch_types=[
        pltpu.VMEM((BPW,), jnp.int32),
        pltpu.VMEM((BPW,), jnp.int32),
        pltpu.VMEM((BPW, A), jnp.float32),
        pltpu.VMEM((BPW, A), jnp.float32),
        pltpu.SemaphoreType.DMA,
        pltpu.SemaphoreType.DMA,
    ],
    compiler_params=_SC_PARAMS,
)
def _embed_gather(table, prev_rel, queries, pe_out, qe_out,
                  idx1_v, idx2_v, rows1_v, rows2_v, sem1, sem2):
    base = _worker_id() * BPW
    pltpu.sync_copy(prev_rel.at[pl.ds(base, BPW)], idx1_v)
    pltpu.sync_copy(queries.at[pl.ds(base, BPW)], idx2_v)
    g1 = pltpu.async_copy(table.at[idx1_v], rows1_v, sem1)
    g2 = pltpu.async_copy(table.at[idx2_v], rows2_v, sem2)
    g1.wait()
    g2.wait()
    pltpu.sync_copy(rows1_v, pe_out.at[pl.ds(base, BPW)])
    pltpu.sync_copy(rows2_v, qe_out.at[pl.ds(base, BPW)])


def _lstm_mlp_body(pe_ref, qe_ref, h_ref, c_ref, wih_t_ref, whh_t_ref, b_ref,
                   w1_t_ref, b1_ref, w2_t_ref, b2_ref,
                   h2_ref, c2_ref, out_ref):
    x = pe_ref[...]
    h = h_ref[...]
    gates = (jnp.dot(x, wih_t_ref[...], preferred_element_type=jnp.float32)
             + jnp.dot(h, whh_t_ref[...], preferred_element_type=jnp.float32)
             + b_ref[...])
    i = jax.nn.sigmoid(gates[:, 0:S])
    f = jax.nn.sigmoid(gates[:, S:2 * S])
    g = jnp.tanh(gates[:, 2 * S:3 * S])
    o = jax.nn.sigmoid(gates[:, 3 * S:4 * S])
    c2 = f * c_ref[...] + i * g
    h2 = o * jnp.tanh(c2)
    sq = jnp.concatenate([h2, qe_ref[...]], axis=-1)
    hid = jax.nn.relu(
        jnp.dot(sq, w1_t_ref[...], preferred_element_type=jnp.float32)
        + b1_ref[...])
    out = jax.nn.relu(
        jnp.dot(hid, w2_t_ref[...], preferred_element_type=jnp.float32)
        + b2_ref[...])
    h2_ref[...] = h2
    c2_ref[...] = c2
    out_ref[...] = out


def _lstm_mlp(pe, qe, h, c, wih_t, whh_t, b, w1_t, b1, w2_t, b2):
    return pl.pallas_call(
        _lstm_mlp_body,
        out_shape=(jax.ShapeDtypeStruct((B, S), jnp.float32),
                   jax.ShapeDtypeStruct((B, S), jnp.float32),
                   jax.ShapeDtypeStruct((B, A), jnp.float32)),
    )(pe, qe, h, c, wih_t, whh_t, b, w1_t, b1, w2_t, b2)


@functools.partial(
    pl.kernel,
    out_type=jax.ShapeDtypeStruct((B, MP), jnp.float32),
    mesh=_MESH,
    scratch_types=[
        pltpu.VMEM((BPW, M), jnp.int32),
        pltpu.VMEM((BPW, A), jnp.float32),
        pltpu.VMEM((NBUF, MP, A), jnp.float32),
        pltpu.VMEM((BPW, MP), jnp.float32),
        pltpu.SemaphoreType.DMA,
        pltpu.SemaphoreType.DMA,
        pltpu.SemaphoreType.DMA,
        pltpu.SemaphoreType.DMA,
    ],
    compiler_params=_SC_PARAMS,
)
def _scores_kernel(table, ids, outv, scores_out,
                   ids_v, out_v, rows_v, sc_v, *sems):
    base = _worker_id() * BPW
    pltpu.sync_copy(ids.at[pl.ds(base, BPW)], ids_v)
    pltpu.sync_copy(outv.at[pl.ds(base, BPW)], out_v)

    def issue(i, b):
        buf = rows_v.at[b]
        pltpu.async_copy(
            table.at[ids_v.at[i, pl.ds(0, MH)]], buf.at[pl.ds(0, MH)],
            sems[b])
        pltpu.async_copy(
            table.at[ids_v.at[i, pl.ds(MH, MH2)]], buf.at[pl.ds(MH, MH2)],
            sems[b])

    def wait(b):
        buf = rows_v.at[b]
        pltpu.make_async_copy(
            table.at[ids_v.at[0, pl.ds(0, MH)]], buf.at[pl.ds(0, MH)],
            sems[b]).wait()
        pltpu.make_async_copy(
            table.at[ids_v.at[0, pl.ds(MH, MH2)]], buf.at[pl.ds(MH, MH2)],
            sems[b]).wait()

    lane = lax.iota(jnp.int32, L)

    def compute(i, b):
        buf = rows_v.at[b]
        o0 = out_v[i, pl.ds(0, L)]
        o1 = out_v[i, pl.ds(L, L)]
        o2 = out_v[i, pl.ds(2 * L, L)]
        o3 = out_v[i, pl.ds(3 * L, L)]

        def blk_body(bi, c2):
            sv = [jnp.zeros((L,), jnp.float32) for _ in range(4)]
            for j in range(L):
                m = bi * L + j
                acc = (buf[m, pl.ds(0, L)] * o0
                       + buf[m, pl.ds(L, L)] * o1
                       + buf[m, pl.ds(2 * L, L)] * o2
                       + buf[m, pl.ds(3 * L, L)] * o3)
                q = j % 4
                sv[q] = jnp.where(lane == j, jnp.sum(acc), sv[q])
            sc_v[i, pl.ds(bi * L, L)] = (sv[0] + sv[1]) + (sv[2] + sv[3])
            return c2

        lax.fori_loop(0, MP // L, blk_body, 0)

    for b in range(NBUF - 1):
        issue(b, b)

    def group_body(k, carry):
        i0 = k * NBUF
        for b in range(NBUF):
            i = i0 + b
            wait(b)
            issue(jnp.minimum(i + NBUF - 1, BPW - 1), (b + NBUF - 1) % NBUF)
            compute(i, b)
        return carry

    lax.fori_loop(0, BPW // NBUF, group_body, 0)
    for b in range(NBUF - 1):
        wait(b)
    pltpu.sync_copy(sc_v, scores_out.at[pl.ds(base, BPW)])


def _logits_body(sc_ref, ids_ref, logits_ref):
    s = sc_ref[...][:, :M]
    s = jnp.where(ids_ref[...] == PAD, -99999.0, s)
    mx = jnp.max(s, axis=-1, keepdims=True)
    lse = jnp.log(jnp.sum(jnp.exp(s - mx), axis=-1, keepdims=True)) + mx
    logits_ref[...] = s - lse


def _logits(scores, rel_ids):
    return pl.pallas_call(
        _logits_body,
        out_shape=jax.ShapeDtypeStruct((B, M), jnp.float32),
    )(scores, rel_ids)


def kernel(prev_state_h, prev_state_c, prev_relation, actions_id, queries,
           rel_emb, W_ih, W_hh, b_ih, b_hh, mlp1_W, mlp1_b, mlp2_W, mlp2_b):
    rel_ids = actions_id[:, :, 0]
    ent_ids = actions_id[:, :, 1]

    pe, qe = _embed_gather(rel_emb, prev_relation, queries)

    b_all = (b_ih + b_hh).reshape(1, 4 * S)
    h2, c2, outv = _lstm_mlp(
        pe, qe, prev_state_h, prev_state_c,
        W_ih.T, W_hh.T, b_all,
        mlp1_W.T, mlp1_b.reshape(1, H), mlp2_W.T, mlp2_b.reshape(1, A))

    scores = _scores_kernel(rel_emb, rel_ids, outv)
    logits = _logits(scores, rel_ids)
    return (logits, rel_ids, ent_ids, h2, c2)


# speedup vs baseline: 2.2312x; 1.0108x over previous
import functools

import jax
import jax.numpy as jnp
from jax import lax
from jax.experimental import pallas as pl
from jax.experimental.pallas import tpu as pltpu
from jax.experimental.pallas import tpu_sc as plsc

B = 4096
M = 200
R = 100000
A = 64
S = 64
H = 128
PAD = 0

L = 16
NC, NS = 2, 16
NW = NC * NS
BPW = B // NW
MP = 208
MH = MP // 2
NBUF = 4
BH = BPW // 2

_MESH = plsc.VectorSubcoreMesh(
    core_axis_name="c", subcore_axis_name="s", num_cores=NC, num_subcores=NS)
_SC_PARAMS = pltpu.CompilerParams(
    use_tc_tiling_on_sc=False, needs_layout_passes=False)


def _worker_id():
    return lax.axis_index("s") * NC + lax.axis_index("c")


@functools.partial(
    pl.kernel,
    out_type=(jax.ShapeDtypeStruct((B, A), jnp.float32),
              jax.ShapeDtypeStruct((B, A), jnp.float32)),
    mesh=_MESH,
    scratch_types=[
        pltpu.VMEM((BPW,), jnp.int32),
        pltpu.VMEM((BPW,), jnp.int32),
        pltpu.VMEM((BPW, A), jnp.float32),
        pltpu.VMEM((BPW, A), jnp.float32),
        pltpu.SemaphoreType.DMA,
        pltpu.SemaphoreType.DMA,
    ],
    compiler_params=_SC_PARAMS,
)
def _embed_gather(table, prev_rel, queries, pe_out, qe_out,
                  idx1_v, idx2_v, rows1_v, rows2_v, sem1, sem2):
    base = _worker_id() * BPW
    pltpu.sync_copy(prev_rel.at[pl.ds(base, BPW)], idx1_v)
    pltpu.sync_copy(queries.at[pl.ds(base, BPW)], idx2_v)
    g1 = pltpu.async_copy(table.at[idx1_v], rows1_v, sem1)
    g2 = pltpu.async_copy(table.at[idx2_v], rows2_v, sem2)
    g1.wait()
    g2.wait()
    pltpu.sync_copy(rows1_v, pe_out.at[pl.ds(base, BPW)])
    pltpu.sync_copy(rows2_v, qe_out.at[pl.ds(base, BPW)])


def _lstm_mlp_body(pe_ref, qe_ref, h_ref, c_ref, wih_t_ref, whh_t_ref, b_ref,
                   w1_t_ref, b1_ref, w2_t_ref, b2_ref,
                   h2_ref, c2_ref, out_ref):
    x = pe_ref[...]
    h = h_ref[...]
    gates = (jnp.dot(x, wih_t_ref[...], preferred_element_type=jnp.float32)
             + jnp.dot(h, whh_t_ref[...], preferred_element_type=jnp.float32)
             + b_ref[...])
    i = jax.nn.sigmoid(gates[:, 0:S])
    f = jax.nn.sigmoid(gates[:, S:2 * S])
    g = jnp.tanh(gates[:, 2 * S:3 * S])
    o = jax.nn.sigmoid(gates[:, 3 * S:4 * S])
    c2 = f * c_ref[...] + i * g
    h2 = o * jnp.tanh(c2)
    sq = jnp.concatenate([h2, qe_ref[...]], axis=-1)
    hid = jax.nn.relu(
        jnp.dot(sq, w1_t_ref[...], preferred_element_type=jnp.float32)
        + b1_ref[...])
    out = jax.nn.relu(
        jnp.dot(hid, w2_t_ref[...], preferred_element_type=jnp.float32)
        + b2_ref[...])
    h2_ref[...] = h2
    c2_ref[...] = c2
    out_ref[...] = out


def _lstm_mlp(pe, qe, h, c, wih_t, whh_t, b, w1_t, b1, w2_t, b2):
    return pl.pallas_call(
        _lstm_mlp_body,
        out_shape=(jax.ShapeDtypeStruct((B, S), jnp.float32),
                   jax.ShapeDtypeStruct((B, S), jnp.float32),
                   jax.ShapeDtypeStruct((B, A), jnp.float32)),
    )(pe, qe, h, c, wih_t, whh_t, b, w1_t, b1, w2_t, b2)


@functools.partial(
    pl.kernel,
    out_type=jax.ShapeDtypeStruct((B, MP), jnp.float32),
    mesh=_MESH,
    scratch_types=[
        pltpu.VMEM((BPW, 2 * M), jnp.int32),
        pltpu.VMEM((MP,), jnp.int32),
        pltpu.VMEM((MP,), jnp.int32),
        pltpu.VMEM((MP,), jnp.int32),
        pltpu.VMEM((MP,), jnp.int32),
        pltpu.VMEM((BPW, A), jnp.float32),
        pltpu.VMEM((NBUF, MP, A), jnp.float32),
        pltpu.VMEM((BH, MP), jnp.float32),
        pltpu.SemaphoreType.DMA,
        pltpu.SemaphoreType.DMA,
        pltpu.SemaphoreType.DMA,
        pltpu.SemaphoreType.DMA,
    ],
    compiler_params=_SC_PARAMS,
)
def _scores_kernel(table, aid, outv, scores_out,
                   aid_v, ix0, ix1, ix2, ix3, out_v, rows_v, sc_v, *sems):
    base = _worker_id() * BPW
    idxs = (ix0, ix1, ix2, ix3)
    pltpu.sync_copy(aid.at[pl.ds(base, BPW)], aid_v)
    pltpu.sync_copy(outv.at[pl.ds(base, BPW)], out_v)
    lane = lax.iota(jnp.int32, L)
    col = [jnp.minimum(2 * L * c + 2 * lane, 2 * M - 2)
           for c in range(MP // L)]

    def issue(i, b):
        row = jnp.full((L,), i, jnp.int32)
        for c in range(MP // L):
            idxs[b][pl.ds(L * c, L)] = plsc.load_gather(aid_v, [row, col[c]])
        buf = rows_v.at[b]
        pltpu.async_copy(
            table.at[idxs[b].at[pl.ds(0, MH)]], buf.at[pl.ds(0, MH)],
            sems[b])
        pltpu.async_copy(
            table.at[idxs[b].at[pl.ds(MH, MH)]], buf.at[pl.ds(MH, MH)],
            sems[b])

    def wait(b):
        buf = rows_v.at[b]
        pltpu.make_async_copy(
            table.at[idxs[b].at[pl.ds(0, MH)]], buf.at[pl.ds(0, MH)],
            sems[b]).wait()
        pltpu.make_async_copy(
            table.at[idxs[b].at[pl.ds(MH, MH)]], buf.at[pl.ds(MH, MH)],
            sems[b]).wait()

    def compute(i, b):
        buf = rows_v.at[b]
        row = jnp.full((L,), i, jnp.int32)
        ih = jnp.bitwise_and(i, BH - 1)
        o0 = out_v[i, pl.ds(0, L)]
        o1 = out_v[i, pl.ds(L, L)]
        o2 = out_v[i, pl.ds(2 * L, L)]
        o3 = out_v[i, pl.ds(3 * L, L)]

        def blk_body(bi, c2):
            sv = [jnp.zeros((L,), jnp.float32) for _ in range(4)]
            for j in range(L):
                m = bi * L + j
                acc = (buf[m, pl.ds(0, L)] * o0
                       + buf[m, pl.ds(L, L)] * o1
                       + buf[m, pl.ds(2 * L, L)] * o2
                       + buf[m, pl.ds(3 * L, L)] * o3)
                q = j % 4
                sv[q] = jnp.where(lane == j, jnp.sum(acc), sv[q])
            idv = plsc.load_gather(
                aid_v, [row, jnp.minimum(2 * L * bi + 2 * lane, 2 * M - 2)])
            merged = (sv[0] + sv[1]) + (sv[2] + sv[3])
            sc_v[ih, pl.ds(bi * L, L)] = jnp.where(
                idv == PAD, -99999.0, merged)
            return c2

        lax.fori_loop(0, MP // L, blk_body, 0)

    for b in range(NBUF - 1):
        issue(b, b)

    def group_body(k, carry):
        i0 = k * NBUF
        for b in range(NBUF):
            i = i0 + b
            wait(b)
            issue(jnp.minimum(i + NBUF - 1, BPW - 1), (b + NBUF - 1) % NBUF)
            compute(i, b)

        @pl.when(k == BH // NBUF - 1)
        def _flush_first_half():
            pltpu.sync_copy(sc_v, scores_out.at[pl.ds(base, BH)])

        return carry

    lax.fori_loop(0, BPW // NBUF, group_body, 0)
    for b in range(NBUF - 1):
        wait(b)
    pltpu.sync_copy(sc_v, scores_out.at[pl.ds(base + BH, BH)])


def _logits_body(sc_ref, logits_ref):
    s = sc_ref[...][:, :M]
    mx = jnp.max(s, axis=-1, keepdims=True)
    lse = jnp.log(jnp.sum(jnp.exp(s - mx), axis=-1, keepdims=True)) + mx
    logits_ref[...] = s - lse


def _logits(scores):
    return pl.pallas_call(
        _logits_body,
        out_shape=jax.ShapeDtypeStruct((B, M), jnp.float32),
    )(scores)


def kernel(prev_state_h, prev_state_c, prev_relation, actions_id, queries,
           rel_emb, W_ih, W_hh, b_ih, b_hh, mlp1_W, mlp1_b, mlp2_W, mlp2_b):
    rel_ids = actions_id[:, :, 0]
    ent_ids = actions_id[:, :, 1]

    pe, qe = _embed_gather(rel_emb, prev_relation, queries)

    b_all = (b_ih + b_hh).reshape(1, 4 * S)
    h2, c2, outv = _lstm_mlp(
        pe, qe, prev_state_h, prev_state_c,
        W_ih.T, W_hh.T, b_all,
        mlp1_W.T, mlp1_b.reshape(1, H), mlp2_W.T, mlp2_b.reshape(1, A))

    scores = _scores_kernel(rel_emb, actions_id.reshape(B, 2 * M), outv)
    logits = _logits(scores)
    return (logits, rel_ids, ent_ids, h2, c2)
